# baseline (speedup 1.0000x reference)
import functools

import jax
import jax.numpy as jnp
from jax import lax
from jax.experimental import pallas as pl
from jax.experimental.pallas import tpu as pltpu
from jax.experimental.pallas import tpu_sc as plsc

B = 16384
F = 26
V = 100000
D = 32
NUM_DIM = 13
FP = 28
G = FP // 4
ROWS2 = B * FP

_NW = 32

_mesh = plsc.VectorSubcoreMesh(core_axis_name="c", subcore_axis_name="s")


WQ = 25088
WC = 3584
VR = 4 * WQ


def _tc_repack_body(x0, x1, x2, x3, out_ref):
    for j, xj in enumerate((x0, x1, x2, x3)):
        out_ref[0, :, j * 32:(j + 1) * 32] = xj[0].T


def _tc_repack(tabT, f0, nf):
    def in_spec(j):
        return pl.BlockSpec((1, D, WC),
                            lambda f, c, j=j: (f0 + f, 0, j * (WQ // WC) + c))

    return pl.pallas_call(
        _tc_repack_body,
        grid=(nf, WQ // WC),
        in_specs=[in_spec(0), in_spec(1), in_spec(2), in_spec(3)],
        out_specs=pl.BlockSpec((1, WC, 128), lambda f, c: (f, c, 0)),
        out_shape=jax.ShapeDtypeStruct((nf, WQ, 128), jnp.float32),
    )(tabT, tabT, tabT, tabT)


_CH = 1024


def _make_sc_gather(nrows):
    per_w = nrows // _NW
    nch = per_w // _CH

    @functools.partial(
        pl.kernel,
        mesh=_mesh,
        out_type=jax.ShapeDtypeStruct((nrows, D), jnp.float32),
        scratch_types=[
            pltpu.VMEM((_CH,), jnp.int32),
            pltpu.VMEM((_CH, D), jnp.float32),
            pltpu.SemaphoreType.DMA,
        ],
        compiler_params=pltpu.CompilerParams(use_tc_tiling_on_sc=False,
                                             needs_layout_passes=False),
    )
    def gather(idx_hbm, tp_hbm, out_hbm, idx0, rows0, si0):
        wid = lax.axis_index("s") * 2 + lax.axis_index("c")
        base = pl.multiple_of(wid * per_w, 1024)

        def chunk_body(c, carry):
            off = pl.multiple_of(base + c * _CH, 1024)
            pltpu.sync_copy(idx_hbm.at[pl.ds(off, _CH)], idx0)
            pltpu.async_copy(tp_hbm.at[idx0], rows0, si0).wait()
            pltpu.sync_copy(rows0, out_hbm.at[pl.ds(off, _CH)])
            return carry

        lax.fori_loop(0, nch, chunk_body, 0)

    return gather


GA = 3
GB = G - GA
_sc_gather_a = _make_sc_gather(GA * B * 4)
_sc_gather_b = _make_sc_gather(GB * B * 4)


def _mlp_body(emba_ref, embb_ref, num_ref, w1g_ref, w1n_ref, b1_ref, w2_ref, b2_ref, out_ref):
    h = jnp.dot(num_ref[...], w1n_ref[...], preferred_element_type=jnp.float32)
    for g in range(GA):
        h = h + jnp.dot(emba_ref[g], w1g_ref[g], preferred_element_type=jnp.float32)
    for g in range(GB):
        h = h + jnp.dot(embb_ref[g], w1g_ref[GA + g], preferred_element_type=jnp.float32)
    h = jnp.maximum(h + b1_ref[...], 0.0)
    o = jnp.dot(h, w2_ref[...], preferred_element_type=jnp.float32)
    out_ref[...] = jnp.maximum(o + b2_ref[...], 0.0)


_BB = 2048


def _mlp(emba, embb, num, w1g, w1n, b1, w2, b2):
    return pl.pallas_call(
        _mlp_body,
        grid=(B // _BB,),
        in_specs=[
            pl.BlockSpec((GA, _BB, 128), lambda i: (0, i, 0)),
            pl.BlockSpec((GB, _BB, 128), lambda i: (0, i, 0)),
            pl.BlockSpec((_BB, NUM_DIM), lambda i: (i, 0)),
            pl.BlockSpec((G, 128, 64), lambda i: (0, 0, 0)),
            pl.BlockSpec((NUM_DIM, 64), lambda i: (0, 0)),
            pl.BlockSpec((1, 64), lambda i: (0, 0)),
            pl.BlockSpec((64, 32), lambda i: (0, 0)),
            pl.BlockSpec((1, 32), lambda i: (0, 0)),
        ],
        out_specs=pl.BlockSpec((_BB, 32), lambda i: (i, 0)),
        out_shape=jax.ShapeDtypeStruct((B, 32), jnp.float32),
    )(emba, embb, num, w1g, w1n, b1, w2, b2)


def kernel(cate_inputs, num_inputs, tables, W1, b1, W2, b2):
    tabT = jnp.transpose(tables, (0, 2, 1))
    nfa = 4 * GA
    tpa = _tc_repack(tabT, 0, nfa)
    tpb = _tc_repack(tabT, nfa, F - nfa)

    f_ar = jnp.arange(FP, dtype=jnp.int32)
    cate_p = jnp.pad(cate_inputs.astype(jnp.int32), ((0, 0), (0, FP - F)))
    cate_r = (cate_p % WQ) * 4 + cate_p // WQ
    bases = jnp.where(f_ar < F, jnp.where(f_ar < nfa, f_ar, f_ar - nfa) * VR, 0)
    idx = (cate_r + bases[None, :]).reshape(B, G, 4)
    idx = jnp.transpose(idx, (1, 0, 2))
    idx_a = idx[:GA].reshape(GA * B * 4)
    idx_b = idx[GA:].reshape(GB * B * 4)

    emb_a = _sc_gather_a(idx_a, tpa.reshape(nfa * VR, D))
    emb_b = _sc_gather_b(idx_b, tpb.reshape((F - nfa) * VR, D))
    emba = emb_a.reshape(GA, B, 128)
    embb = emb_b.reshape(GB, B, 128)

    w1e = W1[:F * D]
    w1g = jnp.concatenate([w1e, jnp.zeros((FP * D - F * D, 64), jnp.float32)]).reshape(G, 128, 64)
    return _mlp(emba, embb, num_inputs, w1g, W1[F * D:], b1.reshape(1, 64),
                W2, b2.reshape(1, 32))


# speedup vs baseline: 1.0296x; 1.0296x over previous
import functools

import jax
import jax.numpy as jnp
from jax import lax
from jax.experimental import pallas as pl
from jax.experimental.pallas import tpu as pltpu
from jax.experimental.pallas import tpu_sc as plsc

B = 16384
F = 26
V = 100000
D = 32
NUM_DIM = 13
FP = 28
G = FP // 4
ROWS2 = B * FP

_NW = 32

_mesh = plsc.VectorSubcoreMesh(core_axis_name="c", subcore_axis_name="s")


WQ = 25088
WC = 3584
VR = 4 * WQ


def _tc_repack_body(x0, x1, x2, x3, out_ref):
    for j, xj in enumerate((x0, x1, x2, x3)):
        out_ref[0, :, j * 32:(j + 1) * 32] = xj[0].T


def _tc_repack(tabT):
    def in_spec(j):
        return pl.BlockSpec((1, D, WC), lambda f, c, j=j: (f, 0, j * (WQ // WC) + c))

    return pl.pallas_call(
        _tc_repack_body,
        grid=(F, WQ // WC),
        in_specs=[in_spec(0), in_spec(1), in_spec(2), in_spec(3)],
        out_specs=pl.BlockSpec((1, WC, 128), lambda f, c: (f, c, 0)),
        out_shape=jax.ShapeDtypeStruct((F, WQ, 128), jnp.float32),
    )(tabT, tabT, tabT, tabT)


_CH = 1024
_PER_W = ROWS2 // _NW
_NCH = _PER_W // _CH


@functools.partial(
    pl.kernel,
    mesh=_mesh,
    out_type=jax.ShapeDtypeStruct((ROWS2, D), jnp.float32),
    scratch_types=[
        pltpu.VMEM((_CH,), jnp.int32),
        pltpu.VMEM((_CH,), jnp.int32),
        pltpu.VMEM((_CH, D), jnp.float32),
        pltpu.VMEM((_CH, D), jnp.float32),
        pltpu.SemaphoreType.DMA,
        pltpu.SemaphoreType.DMA,
        pltpu.SemaphoreType.DMA,
        pltpu.SemaphoreType.DMA,
    ],
    compiler_params=pltpu.CompilerParams(use_tc_tiling_on_sc=False,
                                         needs_layout_passes=False),
)
def _sc_gather(idx_hbm, tp_hbm, out_hbm, idx0, idx1, rows0, rows1,
               si0, si1, so0, so1):
    wid = lax.axis_index("s") * 2 + lax.axis_index("c")
    base = pl.multiple_of(wid * _PER_W, 1024)

    def chunk_body(c, carry):
        off = pl.multiple_of(base + c * _CH, 1024)
        pltpu.sync_copy(idx_hbm.at[pl.ds(off, _CH)], idx0)
        pltpu.async_copy(tp_hbm.at[idx0], rows0, si0).wait()
        pltpu.sync_copy(rows0, out_hbm.at[pl.ds(off, _CH)])
        return carry

    lax.fori_loop(0, _NCH, chunk_body, 0)


def _mlp_body(emb_ref, num_ref, w1g_ref, w1n_ref, b1_ref, w2_ref, b2_ref, out_ref):
    h = jnp.dot(num_ref[...], w1n_ref[...], preferred_element_type=jnp.float32)
    for g in range(G):
        h = h + jnp.dot(emb_ref[g], w1g_ref[g], preferred_element_type=jnp.float32)
    h = jnp.maximum(h + b1_ref[...], 0.0)
    o = jnp.dot(h, w2_ref[...], preferred_element_type=jnp.float32)
    out_ref[...] = jnp.maximum(o + b2_ref[...], 0.0)


_BB = 2048


def _mlp(emb3, num, w1g, w1n, b1, w2, b2):
    return pl.pallas_call(
        _mlp_body,
        grid=(B // _BB,),
        in_specs=[
            pl.BlockSpec((G, _BB, 128), lambda i: (0, i, 0)),
            pl.BlockSpec((_BB, NUM_DIM), lambda i: (i, 0)),
            pl.BlockSpec((G, 128, 64), lambda i: (0, 0, 0)),
            pl.BlockSpec((NUM_DIM, 64), lambda i: (0, 0)),
            pl.BlockSpec((1, 64), lambda i: (0, 0)),
            pl.BlockSpec((64, 32), lambda i: (0, 0)),
            pl.BlockSpec((1, 32), lambda i: (0, 0)),
        ],
        out_specs=pl.BlockSpec((_BB, 32), lambda i: (i, 0)),
        out_shape=jax.ShapeDtypeStruct((B, 32), jnp.float32),
    )(emb3, num, w1g, w1n, b1, w2, b2)


def kernel(cate_inputs, num_inputs, tables, W1, b1, W2, b2):
    tabT = jnp.transpose(tables, (0, 2, 1))
    tp = _tc_repack(tabT)
    tp_rows = tp.reshape(F * VR, D)

    f_ar = jnp.arange(FP, dtype=jnp.int32)
    bases = jnp.where(f_ar < F, f_ar * VR, 0)
    cate_p = jnp.pad(cate_inputs.astype(jnp.int32), ((0, 0), (0, FP - F)))
    cate_r = (cate_p % WQ) * 4 + cate_p // WQ
    idx = (cate_r + bases[None, :]).reshape(B, G, 4)
    idx = jnp.transpose(idx, (1, 0, 2)).reshape(ROWS2)

    emb = _sc_gather(idx, tp_rows)
    emb3 = emb.reshape(G, B, 128)

    w1e = W1[:F * D]
    w1g = jnp.concatenate([w1e, jnp.zeros((FP * D - F * D, 64), jnp.float32)]).reshape(G, 128, 64)
    return _mlp(emb3, num_inputs, w1g, W1[F * D:], b1.reshape(1, 64),
                W2, b2.reshape(1, 32))
